# baseline (speedup 1.0000x reference)
import functools

import numpy as np
import jax
import jax.numpy as jnp
from jax.experimental import pallas as pl
from jax.experimental.pallas import tpu as pltpu


def _gelu(x):
    u = x * (0.7978845608028654 + 0.03567740813636141 * x * x)
    half_x = 0.5 * x
    return half_x + half_x * jnp.tanh(u)


def _spec(h3, mr, mi, ff, gg, ee, cc, m2, kh):
    c, hdim, wdim = h3.shape
    co = mr.shape[1]
    t = jnp.dot(h3.reshape(c * hdim, wdim), ff,
                preferred_element_type=jnp.float32)
    t = t.reshape(c, hdim, 2 * m2).transpose(0, 2, 1)
    u = jnp.dot(t.reshape(c * 2 * m2, hdim), gg,
                preferred_element_type=jnp.float32).reshape(c, 2 * m2, 2 * kh)
    mm = m2 * kh
    ur = (u[:, :m2, :kh] - u[:, m2:, kh:]).reshape(c, mm)
    ui = (u[:, :m2, kh:] + u[:, m2:, :kh]).reshape(c, mm)
    ur2 = jnp.concatenate([ur, ur], axis=-1)
    ui2 = jnp.concatenate([ui, ui], axis=-1)
    zri = jnp.zeros((co, 2 * mm), jnp.float32)
    for i in range(c):
        zri = zri + ur2[i:i + 1] * mr[i] + ui2[i:i + 1] * mi[i]
    z2 = jnp.concatenate([zri[:, :mm].reshape(co, m2, kh),
                          zri[:, mm:].reshape(co, m2, kh)], axis=-1)
    v = jnp.dot(z2.reshape(co * m2, 2 * kh), ee,
                preferred_element_type=jnp.float32)
    v = v.reshape(co, m2, 2 * hdim)
    vr = v[:, :, :hdim].transpose(0, 2, 1)
    vi = v[:, :, hdim:].transpose(0, 2, 1)
    vv = jnp.concatenate([vr, vi], axis=-1)
    out = jnp.dot(vv.reshape(co * hdim, 2 * m2), cc,
                  preferred_element_type=jnp.float32)
    return out.reshape(co, hdim, wdim)


def _block_body(xf_ref, w1_ref, b1_ref, w2_ref, b2_ref,
                mr1_ref, mi1_ref, mr2_ref, mi2_ref,
                ff_ref, gg_ref, ee_ref, cc_ref, o_ref, *, hdim, m2, kh):
    xf = xf_ref[0]
    ci = xf.shape[0]
    wdim = xf.shape[1] // hdim
    co = w1_ref.shape[0]
    hf = _gelu(xf)
    h3 = hf.reshape(ci, hdim, wdim)
    s1 = _spec(h3, mr1_ref[...], mi1_ref[...], ff_ref[...], gg_ref[...],
               ee_ref[...], cc_ref[...], m2, kh)
    c1 = jnp.dot(w1_ref[...], hf,
                 preferred_element_type=jnp.float32) + b1_ref[...]
    out1 = s1 + c1.reshape(co, hdim, wdim)
    h2 = _gelu(out1)
    s2 = _spec(h2, mr2_ref[...], mi2_ref[...], ff_ref[...], gg_ref[...],
               ee_ref[...], cc_ref[...], m2, kh)
    hs = jnp.concatenate([h2.reshape(co, hdim * wdim), xf], axis=0)
    c2 = jnp.dot(w2_ref[...], hs,
                 preferred_element_type=jnp.float32) + b2_ref[...]
    o_ref[0] = s2 + c2.reshape(co, hdim, wdim)


@jax.jit
def _fourier_block(x, f1_w1, f1_w2, conv1_w, conv1_b, f2_w1, f2_w2,
                   conv2_w, conv2_b, sc_w, sc_b):
    bsz, ci, hdim, wdim = x.shape
    co, m1, m2 = f1_w1.shape[1], f1_w1.shape[2], f1_w1.shape[3]
    kh = 2 * m1
    khs = np.concatenate([np.arange(m1), np.arange(hdim - m1, hdim)])
    ang_h = 2.0 * np.pi * np.outer(np.arange(hdim), khs) / hdim
    gg = np.concatenate([np.cos(ang_h), -np.sin(ang_h)], 1).astype(np.float32)
    ang_w = 2.0 * np.pi * np.outer(np.arange(wdim), np.arange(m2)) / wdim
    ff = np.concatenate([np.cos(ang_w), -np.sin(ang_w)], 1).astype(np.float32)
    er = np.cos(ang_h).T / hdim
    ei = np.sin(ang_h).T / hdim
    ee = np.block([[er, ei], [-ei, er]]).astype(np.float32)
    ck = np.full(m2, 2.0 / wdim)
    ck[0] = 1.0 / wdim
    cr = ck[:, None] * np.cos(ang_w).T
    cs = -ck[:, None] * np.sin(ang_w).T
    cc = np.concatenate([cr, cs], 0).astype(np.float32)

    def mix(w_lo, w_hi):
        def part(p):
            m = jnp.concatenate([w_lo[..., p], w_hi[..., p]], axis=2)
            return jnp.transpose(m, (0, 1, 3, 2)).reshape(
                m.shape[0], co, m2 * kh)
        wr, wi = part(0), part(1)
        return (jnp.concatenate([wr, wi], axis=-1),
                jnp.concatenate([-wi, wr], axis=-1))

    mr1, mi1 = mix(f1_w1, f1_w2)
    mr2, mi2 = mix(f2_w1, f2_w2)
    b1 = conv1_b[:, None]
    b2 = (conv2_b + sc_b)[:, None]
    w2cat = jnp.concatenate([conv2_w, sc_w], axis=1)
    xf = x.reshape(bsz, ci, hdim * wdim)

    resident = pl.BlockSpec(memory_space=pltpu.MemorySpace.VMEM)
    out = pl.pallas_call(
        functools.partial(_block_body, hdim=hdim, m2=m2, kh=kh),
        grid=(bsz,),
        in_specs=[pl.BlockSpec((1, ci, hdim * wdim), lambda b: (b, 0, 0))]
                 + [resident] * 12,
        out_specs=pl.BlockSpec((1, co, hdim, wdim), lambda b: (b, 0, 0, 0)),
        out_shape=jax.ShapeDtypeStruct((bsz, co, hdim, wdim), jnp.float32),
        compiler_params=pltpu.CompilerParams(
            dimension_semantics=("parallel",)),
    )(xf, conv1_w, b1, w2cat, b2, mr1, mi1, mr2, mi2,
      jnp.asarray(ff), jnp.asarray(gg), jnp.asarray(ee), jnp.asarray(cc))
    return out


def kernel(x, f1_w1, f1_w2, conv1_w, conv1_b, f2_w1, f2_w2,
           conv2_w, conv2_b, sc_w, sc_b):
    return _fourier_block(x, f1_w1, f1_w2, conv1_w, conv1_b,
                          f2_w1, f2_w2, conv2_w, conv2_b, sc_w, sc_b)


# speedup vs baseline: 1.0007x; 1.0007x over previous
import functools

import numpy as np
import jax
import jax.numpy as jnp
from jax.experimental import pallas as pl
from jax.experimental.pallas import tpu as pltpu


def _gelu(x):
    u = x * (0.7978845608028654 + 0.03567740813636141 * x * x)
    half_x = 0.5 * x
    return half_x + half_x * jnp.tanh(u)


def _spec(h3, mr, mi, ff, gg, ee, cc, m2, kh):
    c, hdim, wdim = h3.shape
    co = mr.shape[1]
    t = jnp.dot(h3.reshape(c * hdim, wdim), ff,
                preferred_element_type=jnp.float32)
    t = t.reshape(c, hdim, 2 * m2).transpose(0, 2, 1)
    u = jnp.dot(t.reshape(c * 2 * m2, hdim), gg,
                preferred_element_type=jnp.float32).reshape(c, 2 * m2, 2 * kh)
    mm = m2 * kh
    ur = (u[:, :m2, :kh] - u[:, m2:, kh:]).reshape(c, mm)
    ui = (u[:, :m2, kh:] + u[:, m2:, :kh]).reshape(c, mm)
    ur2 = jnp.concatenate([ur, ur], axis=-1)
    ui2 = jnp.concatenate([ui, ui], axis=-1)
    zri = jnp.zeros((co, 2 * mm), jnp.float32)
    for i in range(c):
        zri = zri + ur2[i:i + 1] * mr[i] + ui2[i:i + 1] * mi[i]
    z2 = jnp.concatenate([zri[:, :mm].reshape(co, m2, kh),
                          zri[:, mm:].reshape(co, m2, kh)], axis=-1)
    v = jnp.dot(z2.reshape(co * m2, 2 * kh), ee,
                preferred_element_type=jnp.float32)
    v = v.reshape(co, m2, 2 * hdim)
    vr = v[:, :, :hdim].transpose(0, 2, 1)
    vi = v[:, :, hdim:].transpose(0, 2, 1)
    vv = jnp.concatenate([vr, vi], axis=-1)
    out = jnp.dot(vv.reshape(co * hdim, 2 * m2), cc,
                  preferred_element_type=jnp.float32)
    return out.reshape(co, hdim, wdim)


def _block_body(xf_ref, w1_ref, b1_ref, w2_ref, b2_ref,
                mr1_ref, mi1_ref, mr2_ref, mi2_ref,
                ff_ref, gg_ref, ee_ref, cc_ref, o_ref, *, hdim, m2, kh):
    xf = xf_ref[0]
    ci = xf.shape[0]
    wdim = xf.shape[1] // hdim
    co = w1_ref.shape[0]
    hf = _gelu(xf)
    h3 = hf.reshape(ci, hdim, wdim)
    s1 = _spec(h3, mr1_ref[...], mi1_ref[...], ff_ref[...], gg_ref[...],
               ee_ref[...], cc_ref[...], m2, kh)
    c1 = jnp.dot(w1_ref[...], hf,
                 preferred_element_type=jnp.float32) + b1_ref[...]
    out1 = s1 + c1.reshape(co, hdim, wdim)
    h2 = _gelu(out1)
    s2 = _spec(h2, mr2_ref[...], mi2_ref[...], ff_ref[...], gg_ref[...],
               ee_ref[...], cc_ref[...], m2, kh)
    hs = jnp.concatenate([h2.reshape(co, hdim * wdim), xf], axis=0)
    c2 = jnp.dot(w2_ref[...], hs,
                 preferred_element_type=jnp.float32) + b2_ref[...]
    o_ref[0] = s2 + c2.reshape(co, hdim, wdim)


@jax.jit
def _fourier_block(x, f1_w1, f1_w2, conv1_w, conv1_b, f2_w1, f2_w2,
                   conv2_w, conv2_b, sc_w, sc_b):
    bsz, ci, hdim, wdim = x.shape
    co, m1, m2 = f1_w1.shape[1], f1_w1.shape[2], f1_w1.shape[3]
    kh = 2 * m1
    khs = np.concatenate([np.arange(m1), np.arange(hdim - m1, hdim)])
    ang_h = 2.0 * np.pi * np.outer(np.arange(hdim), khs) / hdim
    gg = np.concatenate([np.cos(ang_h), -np.sin(ang_h)], 1).astype(np.float32)
    ang_w = 2.0 * np.pi * np.outer(np.arange(wdim), np.arange(m2)) / wdim
    ff = np.concatenate([np.cos(ang_w), -np.sin(ang_w)], 1).astype(np.float32)
    er = np.cos(ang_h).T / hdim
    ei = np.sin(ang_h).T / hdim
    ee = np.block([[er, ei], [-ei, er]]).astype(np.float32)
    ck = np.full(m2, 2.0 / wdim)
    ck[0] = 1.0 / wdim
    cr = ck[:, None] * np.cos(ang_w).T
    cs = -ck[:, None] * np.sin(ang_w).T
    cc = np.concatenate([cr, cs], 0).astype(np.float32)

    def mix(w_lo, w_hi):
        def part(p):
            m = jnp.concatenate([w_lo[..., p], w_hi[..., p]], axis=2)
            return jnp.transpose(m, (0, 1, 3, 2)).reshape(
                m.shape[0], co, m2 * kh)
        wr, wi = part(0), part(1)
        return (jnp.concatenate([wr, wi], axis=-1),
                jnp.concatenate([-wi, wr], axis=-1))

    mr1, mi1 = mix(f1_w1, f1_w2)
    mr2, mi2 = mix(f2_w1, f2_w2)
    b1 = conv1_b[:, None]
    b2 = (conv2_b + sc_b)[:, None]
    w2cat = jnp.concatenate([conv2_w, sc_w], axis=1)
    xf = x.reshape(bsz, ci, hdim * wdim)

    full = lambda s: pl.BlockSpec(s, lambda b: (0,) * len(s))
    out = pl.pallas_call(
        functools.partial(_block_body, hdim=hdim, m2=m2, kh=kh),
        grid=(bsz,),
        in_specs=[
            pl.BlockSpec((1, ci, hdim * wdim), lambda b: (b, 0, 0)),
            full((co, ci)),
            full((co, 1)),
            full((co, co + ci)),
            full((co, 1)),
            full((ci, co, 2 * m2 * kh)),
            full((ci, co, 2 * m2 * kh)),
            full((co, co, 2 * m2 * kh)),
            full((co, co, 2 * m2 * kh)),
            full((wdim, 2 * m2)),
            full((hdim, 2 * kh)),
            full((2 * kh, 2 * hdim)),
            full((2 * m2, wdim)),
        ],
        out_specs=pl.BlockSpec((1, co, hdim, wdim), lambda b: (b, 0, 0, 0)),
        out_shape=jax.ShapeDtypeStruct((bsz, co, hdim, wdim), jnp.float32),
        compiler_params=pltpu.CompilerParams(
            dimension_semantics=("parallel",)),
    )(xf, conv1_w, b1, w2cat, b2, mr1, mi1, mr2, mi2,
      jnp.asarray(ff), jnp.asarray(gg), jnp.asarray(ee), jnp.asarray(cc))
    return out


def kernel(x, f1_w1, f1_w2, conv1_w, conv1_b, f2_w1, f2_w2,
           conv2_w, conv2_b, sc_w, sc_b):
    return _fourier_block(x, f1_w1, f1_w2, conv1_w, conv1_b,
                          f2_w1, f2_w2, conv2_w, conv2_b, sc_w, sc_b)


# speedup vs baseline: 1.0657x; 1.0649x over previous
import functools

import numpy as np
import jax
import jax.numpy as jnp
from jax.experimental import pallas as pl
from jax.experimental.pallas import tpu as pltpu


def _gelu(x):
    u = x * (0.7978845608028654 + 0.03567740813636141 * x * x)
    half_x = 0.5 * x
    return half_x + half_x * jnp.tanh(u)


def _spec(h3, mr, mi, ff, gg, ee, cc, m2, kh):
    c, hdim, wdim = h3.shape
    co = mr.shape[1]
    t = jnp.dot(h3.reshape(c * hdim, wdim), ff,
                preferred_element_type=jnp.float32)
    t = t.reshape(c, hdim, 2 * m2).transpose(0, 2, 1)
    u = jnp.dot(t.reshape(c * 2 * m2, hdim), gg,
                preferred_element_type=jnp.float32).reshape(c, 2 * m2, 2 * kh)
    mm = m2 * kh
    ur = (u[:, :m2, :kh] - u[:, m2:, kh:]).reshape(c, mm)
    ui = (u[:, :m2, kh:] + u[:, m2:, :kh]).reshape(c, mm)
    zr = jnp.zeros((co, mm), jnp.float32)
    zi = jnp.zeros((co, mm), jnp.float32)
    for i in range(c):
        a = ur[i:i + 1]
        b = ui[i:i + 1]
        wr = mr[i]
        wi = mi[i]
        zr = zr + a * wr - b * wi
        zi = zi + a * wi + b * wr
    z2 = jnp.concatenate([zr.reshape(co, m2, kh),
                          zi.reshape(co, m2, kh)], axis=-1)
    v = jnp.dot(z2.reshape(co * m2, 2 * kh), ee,
                preferred_element_type=jnp.float32)
    v = v.reshape(co, m2, 2 * hdim)
    vr = v[:, :, :hdim].transpose(0, 2, 1)
    vi = v[:, :, hdim:].transpose(0, 2, 1)
    vv = jnp.concatenate([vr, vi], axis=-1)
    out = jnp.dot(vv.reshape(co * hdim, 2 * m2), cc,
                  preferred_element_type=jnp.float32)
    return out.reshape(co, hdim, wdim)


def _block_body(xf_ref, w1_ref, b1_ref, w2_ref, b2_ref,
                mr1_ref, mi1_ref, mr2_ref, mi2_ref,
                ff_ref, gg_ref, ee_ref, cc_ref, o_ref, *, hdim, m2, kh):
    xf = xf_ref[0]
    ci = xf.shape[0]
    wdim = xf.shape[1] // hdim
    co = w1_ref.shape[0]
    hf = _gelu(xf)
    h3 = hf.reshape(ci, hdim, wdim)
    s1 = _spec(h3, mr1_ref[...], mi1_ref[...], ff_ref[...], gg_ref[...],
               ee_ref[...], cc_ref[...], m2, kh)
    c1 = jnp.dot(w1_ref[...], hf,
                 preferred_element_type=jnp.float32) + b1_ref[...]
    out1 = s1 + c1.reshape(co, hdim, wdim)
    h2 = _gelu(out1)
    s2 = _spec(h2, mr2_ref[...], mi2_ref[...], ff_ref[...], gg_ref[...],
               ee_ref[...], cc_ref[...], m2, kh)
    hs = jnp.concatenate([h2.reshape(co, hdim * wdim), xf], axis=0)
    c2 = jnp.dot(w2_ref[...], hs,
                 preferred_element_type=jnp.float32) + b2_ref[...]
    o_ref[0] = s2 + c2.reshape(co, hdim, wdim)


@jax.jit
def _fourier_block(x, f1_w1, f1_w2, conv1_w, conv1_b, f2_w1, f2_w2,
                   conv2_w, conv2_b, sc_w, sc_b):
    bsz, ci, hdim, wdim = x.shape
    co, m1, m2 = f1_w1.shape[1], f1_w1.shape[2], f1_w1.shape[3]
    kh = 2 * m1
    khs = np.concatenate([np.arange(m1), np.arange(hdim - m1, hdim)])
    ang_h = 2.0 * np.pi * np.outer(np.arange(hdim), khs) / hdim
    gg = np.concatenate([np.cos(ang_h), -np.sin(ang_h)], 1).astype(np.float32)
    ang_w = 2.0 * np.pi * np.outer(np.arange(wdim), np.arange(m2)) / wdim
    ff = np.concatenate([np.cos(ang_w), -np.sin(ang_w)], 1).astype(np.float32)
    er = np.cos(ang_h).T / hdim
    ei = np.sin(ang_h).T / hdim
    ee = np.block([[er, ei], [-ei, er]]).astype(np.float32)
    ck = np.full(m2, 2.0 / wdim)
    ck[0] = 1.0 / wdim
    cr = ck[:, None] * np.cos(ang_w).T
    cs = -ck[:, None] * np.sin(ang_w).T
    cc = np.concatenate([cr, cs], 0).astype(np.float32)

    def mix(w_lo, w_hi, part):
        m = jnp.concatenate([w_lo[..., part], w_hi[..., part]], axis=2)
        return jnp.transpose(m, (0, 1, 3, 2)).reshape(m.shape[0], co, m2 * kh)

    mr1, mi1 = mix(f1_w1, f1_w2, 0), mix(f1_w1, f1_w2, 1)
    mr2, mi2 = mix(f2_w1, f2_w2, 0), mix(f2_w1, f2_w2, 1)
    b1 = conv1_b[:, None]
    b2 = (conv2_b + sc_b)[:, None]
    w2cat = jnp.concatenate([conv2_w, sc_w], axis=1)
    xf = x.reshape(bsz, ci, hdim * wdim)

    full = lambda s: pl.BlockSpec(s, lambda b: (0,) * len(s))
    out = pl.pallas_call(
        functools.partial(_block_body, hdim=hdim, m2=m2, kh=kh),
        grid=(bsz,),
        in_specs=[
            pl.BlockSpec((1, ci, hdim * wdim), lambda b: (b, 0, 0)),
            full((co, ci)),
            full((co, 1)),
            full((co, co + ci)),
            full((co, 1)),
            full((ci, co, m2 * kh)),
            full((ci, co, m2 * kh)),
            full((co, co, m2 * kh)),
            full((co, co, m2 * kh)),
            full((wdim, 2 * m2)),
            full((hdim, 2 * kh)),
            full((2 * kh, 2 * hdim)),
            full((2 * m2, wdim)),
        ],
        out_specs=pl.BlockSpec((1, co, hdim, wdim), lambda b: (b, 0, 0, 0)),
        out_shape=jax.ShapeDtypeStruct((bsz, co, hdim, wdim), jnp.float32),
        compiler_params=pltpu.CompilerParams(
            dimension_semantics=("parallel",)),
    )(xf, conv1_w, b1, w2cat, b2, mr1, mi1, mr2, mi2,
      jnp.asarray(ff), jnp.asarray(gg), jnp.asarray(ee), jnp.asarray(cc))
    return out


def kernel(x, f1_w1, f1_w2, conv1_w, conv1_b, f2_w1, f2_w2,
           conv2_w, conv2_b, sc_w, sc_b):
    return _fourier_block(x, f1_w1, f1_w2, conv1_w, conv1_b,
                          f2_w1, f2_w2, conv2_w, conv2_b, sc_w, sc_b)
